# baseline (speedup 1.0000x reference)
import jax
import jax.numpy as jnp
from jax import lax
from jax.experimental import pallas as pl
from jax.experimental.pallas import tpu as pltpu
from jax.experimental.pallas import tpu_sc as plsc

B, V, K, D = 16, 50000, 32, 3
BD = B * D
BDP = 64
NC, NS = 2, 16
NW = NC * NS
G = 32
R0 = 100
R1 = 0
VPAD = NS * G * (R0 + R1)
GK = G * K
IDX_ROWS = GK // 128
L = 16


def _splat(wsrc, k):
    return lax.gather(
        wsrc, jnp.full((L, 1), k % L, jnp.int32),
        dimension_numbers=lax.GatherDimensionNumbers(
            offset_dims=(), collapsed_slice_dims=(0,),
            start_index_map=(0,)),
        slice_sizes=(1,),
        mode=lax.GatherScatterMode.PROMISE_IN_BOUNDS)


def _body(xb_hbm, idx_hbm, w_hbm, out_hbm,
          idx_v, w_v, own_v, rows_v, sq_v,
          sem_g0, sem_g1, sem_i0, sem_i1, sem_w0, sem_w1):
    sid = lax.axis_index("subcore")
    cid = lax.axis_index("core")
    wid = cid * NS + sid
    Rc = jnp.where(cid == 0, R0, R1)
    vbase = jnp.where(cid == 0, sid * (G * R0),
                      NS * G * R0 + sid * (G * R1))
    irow0 = vbase * K // 128
    sem_g = (sem_g0, sem_g1)
    sem_i = (sem_i0, sem_i1)
    sem_w = (sem_w0, sem_w1)

    for c in range(4):
        sq_v[pl.ds(c * L, L)] = jnp.zeros((L,), jnp.float32)

    def fire_idx(r, buf):
        return pltpu.async_copy(
            idx_hbm.at[pl.ds(irow0 + r * IDX_ROWS, IDX_ROWS)],
            idx_v.at[buf], sem_i[buf])

    def fire_wown(r, buf):
        v0 = vbase + r * G
        pltpu.async_copy(w_hbm.at[pl.ds(v0, G)], w_v.at[buf], sem_w[buf])
        pltpu.async_copy(xb_hbm.at[pl.ds(v0, G)], own_v.at[buf], sem_w[buf])

    def wait_wown(buf):
        pltpu.make_async_copy(w_hbm.at[pl.ds(0, G)], w_v.at[buf],
                              sem_w[buf]).wait()
        pltpu.make_async_copy(xb_hbm.at[pl.ds(0, G)], own_v.at[buf],
                              sem_w[buf]).wait()

    def fire_gathers(buf):
        for j in range(IDX_ROWS):
            pltpu.async_copy(
                xb_hbm.at[idx_v.at[buf, j]],
                rows_v.at[buf, pl.ds(j * 128, 128)], sem_g[buf])

    def wait_gathers(buf):
        for j in range(IDX_ROWS):
            pltpu.make_async_copy(
                xb_hbm.at[idx_v.at[buf, j]],
                rows_v.at[buf, pl.ds(j * 128, 128)], sem_g[buf]).wait()

    def compute(buf):
        @pl.loop(0, G)
        def _(vl):
            a0 = own_v[buf, vl, 0:32]
            a1 = own_v[buf, vl, 32:64]
            w0 = w_v[buf, vl, 0:16]
            w1 = w_v[buf, vl, 16:32]
            for k in range(K):
                wk = _splat(w0 if k < L else w1, k)
                wp = plsc.pack(wk, wk, format=plsc.PackFormat.INTERLEAVED)
                row = vl * K + k
                a0 = a0 + wp * rows_v[buf, row, 0:32]
                a1 = a1 + wp * rows_v[buf, row, 32:64]
            f0, f1 = plsc.unpack(a0, format=plsc.PackFormat.INTERLEAVED,
                                 preferred_element_type=jnp.float32)
            f2, f3 = plsc.unpack(a1, format=plsc.PackFormat.INTERLEAVED,
                                 preferred_element_type=jnp.float32)
            sq_v[0:16] += f0 * f0
            sq_v[16:32] += f1 * f1
            sq_v[32:48] += f2 * f2
            sq_v[48:64] += f3 * f3

    @pl.when(Rc > 0)
    def _():
        fire_idx(0, 0).wait()
        fire_wown(0, 0)
        fire_gathers(0)

    @pl.when(Rc > 1)
    def _():
        fire_idx(1, 1)

    @pl.loop(0, Rc, step=2)
    def _(r2):
      for s in range(2):
        r = r2 + s
        buf = s
        nb = 1 - buf

        @pl.when(r + 1 < Rc)
        def _():
            pltpu.make_async_copy(
                idx_hbm.at[pl.ds(0, IDX_ROWS)], idx_v.at[nb],
                sem_i[nb]).wait()
            fire_gathers(nb)
        wait_gathers(buf)

        @pl.when(r + 2 < Rc)
        def _():
            fire_idx(r + 2, buf)

        @pl.when(r + 1 < Rc)
        def _():
            fire_wown(r + 1, nb)
        wait_wown(buf)
        compute(buf)

    part = (sq_v[0:16] + sq_v[16:32]) + (sq_v[32:48] + sq_v[48:64])
    sq_v[0:16] = part
    pltpu.sync_copy(sq_v.at[pl.ds(0, 16)], out_hbm.at[wid])


@jax.jit
def kernel(x, targets, nbs_idxs, nbs_weights):
    del targets
    x2 = x.transpose(1, 0, 2).reshape(V, BD)
    xb = jnp.pad(x2.astype(jnp.bfloat16),
                 ((0, VPAD - V), (0, BDP - BD)))
    idx = nbs_idxs.astype(jnp.int32)
    idx = jnp.pad(idx, ((0, VPAD - V), (0, 0))).reshape(-1, 128)
    w = jnp.pad(nbs_weights, ((0, VPAD - V), (0, 0)))

    mesh = plsc.VectorSubcoreMesh(core_axis_name="core",
                                  subcore_axis_name="subcore")
    run = pl.kernel(
        _body,
        out_type=jax.ShapeDtypeStruct((NW, L), jnp.float32),
        mesh=mesh,
        scratch_types=[
            pltpu.VMEM((2, IDX_ROWS, 128), jnp.int32),
            pltpu.VMEM((2, G, K), jnp.float32),
            pltpu.VMEM((2, G, BDP), jnp.bfloat16),
            pltpu.VMEM((2, GK, BDP), jnp.bfloat16),
            pltpu.VMEM((BDP,), jnp.float32),
            pltpu.SemaphoreType.DMA,
            pltpu.SemaphoreType.DMA,
            pltpu.SemaphoreType.DMA,
            pltpu.SemaphoreType.DMA,
            pltpu.SemaphoreType.DMA,
            pltpu.SemaphoreType.DMA,
        ],
        compiler_params=pltpu.CompilerParams(use_tc_tiling_on_sc=False,
                                             needs_layout_passes=False),
    )
    parts = run(xb, idx, w)
    return jnp.sum(parts) / (B * V * D)


# speedup vs baseline: 1.6735x; 1.6735x over previous
import jax
import jax.numpy as jnp
from jax import lax
from jax.experimental import pallas as pl
from jax.experimental.pallas import tpu as pltpu
from jax.experimental.pallas import tpu_sc as plsc

B, V, K, D = 16, 50000, 32, 3
BD = B * D
BDP = 64
NC, NS = 2, 16
NW = NC * NS
G = 32
R0 = 50
R1 = 50
VPAD = NS * G * (R0 + R1)
GK = G * K
IDX_ROWS = GK // 128
L = 16


def _splat(wsrc, k):
    return lax.gather(
        wsrc, jnp.full((L, 1), k % L, jnp.int32),
        dimension_numbers=lax.GatherDimensionNumbers(
            offset_dims=(), collapsed_slice_dims=(0,),
            start_index_map=(0,)),
        slice_sizes=(1,),
        mode=lax.GatherScatterMode.PROMISE_IN_BOUNDS)


def _body(xb_hbm, idx_hbm, w_hbm, out_hbm,
          idx_v, w_v, own_v, rows_v, sq_v,
          sem_g0, sem_g1, sem_i0, sem_i1, sem_w0, sem_w1):
    sid = lax.axis_index("subcore")
    cid = lax.axis_index("core")
    wid = cid * NS + sid
    Rc = jnp.where(cid == 0, R0, R1)
    vbase = jnp.where(cid == 0, sid * (G * R0),
                      NS * G * R0 + sid * (G * R1))
    irow0 = vbase * K // 128
    sem_g = (sem_g0, sem_g1)
    sem_i = (sem_i0, sem_i1)
    sem_w = (sem_w0, sem_w1)

    for c in range(4):
        sq_v[pl.ds(c * L, L)] = jnp.zeros((L,), jnp.float32)

    def fire_idx(r, buf):
        return pltpu.async_copy(
            idx_hbm.at[pl.ds(irow0 + r * IDX_ROWS, IDX_ROWS)],
            idx_v.at[buf], sem_i[buf])

    def fire_wown(r, buf):
        v0 = vbase + r * G
        pltpu.async_copy(w_hbm.at[pl.ds(v0, G)], w_v.at[buf], sem_w[buf])
        pltpu.async_copy(xb_hbm.at[pl.ds(v0, G)], own_v.at[buf], sem_w[buf])

    def wait_wown(buf):
        pltpu.make_async_copy(w_hbm.at[pl.ds(0, G)], w_v.at[buf],
                              sem_w[buf]).wait()
        pltpu.make_async_copy(xb_hbm.at[pl.ds(0, G)], own_v.at[buf],
                              sem_w[buf]).wait()

    def fire_gathers(buf):
        for j in range(IDX_ROWS):
            pltpu.async_copy(
                xb_hbm.at[idx_v.at[buf, j]],
                rows_v.at[buf, pl.ds(j * 128, 128)], sem_g[buf])

    def wait_gathers(buf):
        for j in range(IDX_ROWS):
            pltpu.make_async_copy(
                xb_hbm.at[idx_v.at[buf, j]],
                rows_v.at[buf, pl.ds(j * 128, 128)], sem_g[buf]).wait()

    def compute(buf):
        @pl.loop(0, G)
        def _(vl):
            a0, a1 = plsc.unpack(own_v[buf, vl, 0:64],
                                 format=plsc.PackFormat.INTERLEAVED,
                                 preferred_element_type=jnp.bfloat16)
            w0 = w_v[buf, vl, 0:16]
            w1 = w_v[buf, vl, 16:32]
            for k in range(K):
                wk = _splat(w0 if k < L else w1, k)
                wp = plsc.pack(wk, wk, format=plsc.PackFormat.INTERLEAVED)
                row = vl * K + k
                r0, r1 = plsc.unpack(rows_v[buf, row, 0:64],
                                     format=plsc.PackFormat.INTERLEAVED,
                                     preferred_element_type=jnp.bfloat16)
                a0 = a0 + wp * r0
                a1 = a1 + wp * r1
            f0, f1 = plsc.unpack(a0, format=plsc.PackFormat.INTERLEAVED,
                                 preferred_element_type=jnp.float32)
            f2, f3 = plsc.unpack(a1, format=plsc.PackFormat.INTERLEAVED,
                                 preferred_element_type=jnp.float32)
            sq_v[0:16] += f0 * f0
            sq_v[16:32] += f1 * f1
            sq_v[32:48] += f2 * f2
            sq_v[48:64] += f3 * f3

    @pl.when(Rc > 0)
    def _():
        fire_idx(0, 0).wait()
        fire_wown(0, 0)
        fire_gathers(0)

    @pl.when(Rc > 1)
    def _():
        fire_idx(1, 1)

    @pl.loop(0, Rc, step=2)
    def _(r2):
      for s in range(2):
        r = r2 + s
        buf = s
        nb = 1 - buf

        @pl.when(r + 1 < Rc)
        def _():
            pltpu.make_async_copy(
                idx_hbm.at[pl.ds(0, IDX_ROWS)], idx_v.at[nb],
                sem_i[nb]).wait()
            fire_gathers(nb)
        wait_gathers(buf)

        @pl.when(r + 2 < Rc)
        def _():
            fire_idx(r + 2, buf)

        @pl.when(r + 1 < Rc)
        def _():
            fire_wown(r + 1, nb)
        wait_wown(buf)
        compute(buf)

    part = (sq_v[0:16] + sq_v[16:32]) + (sq_v[32:48] + sq_v[48:64])
    sq_v[0:16] = part
    pltpu.sync_copy(sq_v.at[pl.ds(0, 16)], out_hbm.at[wid])


@jax.jit
def kernel(x, targets, nbs_idxs, nbs_weights):
    del targets
    x2 = x.transpose(1, 0, 2).reshape(V, BD)
    xb = jnp.pad(x2.astype(jnp.float8_e4m3fn),
                 ((0, VPAD - V), (0, BDP - BD)))
    idx = nbs_idxs.astype(jnp.int32)
    idx = jnp.pad(idx, ((0, VPAD - V), (0, 0))).reshape(-1, 128)
    w = jnp.pad(nbs_weights, ((0, VPAD - V), (0, 0)))

    mesh = plsc.VectorSubcoreMesh(core_axis_name="core",
                                  subcore_axis_name="subcore")
    run = pl.kernel(
        _body,
        out_type=jax.ShapeDtypeStruct((NW, L), jnp.float32),
        mesh=mesh,
        scratch_types=[
            pltpu.VMEM((2, IDX_ROWS, 128), jnp.int32),
            pltpu.VMEM((2, G, K), jnp.float32),
            pltpu.VMEM((2, G, BDP), jnp.float8_e4m3fn),
            pltpu.VMEM((2, GK, BDP), jnp.float8_e4m3fn),
            pltpu.VMEM((BDP,), jnp.float32),
            pltpu.SemaphoreType.DMA,
            pltpu.SemaphoreType.DMA,
            pltpu.SemaphoreType.DMA,
            pltpu.SemaphoreType.DMA,
            pltpu.SemaphoreType.DMA,
            pltpu.SemaphoreType.DMA,
        ],
        compiler_params=pltpu.CompilerParams(use_tc_tiling_on_sc=False,
                                             needs_layout_passes=False),
    )
    parts = run(xb, idx, w)
    return jnp.sum(parts) / (B * V * D)
